# baseline (speedup 1.0000x reference)
import functools

import jax
import jax.numpy as jnp
from jax import lax
from jax.experimental import pallas as pl
from jax.experimental.pallas import tpu as pltpu
from jax.experimental.pallas import tpu_sc as plsc

B = 4
S = 2048
VOCAB = 100000
MAX_POS = 2048
H = 1024

NC = 2
NS = 16
NW = NC * NS
N_TOK = B * S
CHUNK = 16
LANES = 16
KPER = H // LANES
NSPLIT = 2

_mesh = plsc.VectorSubcoreMesh(core_axis_name="c", subcore_axis_name="s")


def _make_emb_lookup(n_tok):
    tok_per_w = n_tok // NW
    nchunk = tok_per_w // CHUNK

    @functools.partial(
        pl.kernel,
        mesh=_mesh,
        out_type=jax.ShapeDtypeStruct((n_tok, H), jnp.float32),
        scratch_types=[
            pltpu.VMEM((tok_per_w,), jnp.int32),
            pltpu.VMEM((tok_per_w,), jnp.int32),
            pltpu.VMEM((4, CHUNK, H), jnp.float32),
            pltpu.VMEM((3, CHUNK, H), jnp.float32),
            pltpu.SemaphoreType.DMA,
            pltpu.SemaphoreType.DMA,
            pltpu.SemaphoreType.DMA,
            pltpu.SemaphoreType.DMA,
            pltpu.SemaphoreType.DMA,
            pltpu.SemaphoreType.DMA,
            pltpu.SemaphoreType.DMA,
            pltpu.SemaphoreType.DMA,
        ],
    )
    def _emb_lookup(wtab, ptab, wids, pids, out, widx_v, pidx_v, wbuf, pbuf,
                    wsem0, wsem1, wsem2, psem0, psem1, psem2, osem0, osem1):
        wsem = (wsem0, wsem1, wsem2)
        psem = (psem0, psem1, psem2)
        osem = (osem0, osem1)
        w = lax.axis_index("s") * NC + lax.axis_index("c")
        base = w * tok_per_w

        pltpu.sync_copy(wids.at[pl.ds(base, tok_per_w)], widx_v)
        pltpu.sync_copy(pids.at[pl.ds(base, tok_per_w)], pidx_v)

        def issue_gather(i):
            wcopy = pltpu.async_copy(
                wtab.at[widx_v.at[pl.ds(i * CHUNK, CHUNK)]], wbuf.at[i % 4],
                wsem[i % 3])
            pcopy = pltpu.async_copy(
                ptab.at[pidx_v.at[pl.ds(i * CHUNK, CHUNK)]], pbuf.at[i % 3],
                psem[i % 3])
            return wcopy, pcopy

        pending_g = {0: issue_gather(0), 1: issue_gather(1)}
        pending_o = {}

        for i in range(nchunk):
            ws, ps = i % 4, i % 3
            if i + 2 < nchunk:
                if i - 2 in pending_o:
                    pending_o.pop(i - 2).wait()
                pending_g[i + 2] = issue_gather(i + 2)
            wc, pc = pending_g.pop(i)
            wc.wait()
            pc.wait()

            def row_body(r, _, _ws=ws, _ps=ps):
                for k in range(KPER):
                    x = pbuf[_ps, r, pl.ds(k * LANES, LANES)]
                    plsc.addupdate(wbuf.at[_ws, r, pl.ds(k * LANES, LANES)], x)
                return 0

            lax.fori_loop(0, CHUNK, row_body, 0, unroll=False)

            pending_o[i] = pltpu.async_copy(
                wbuf.at[ws], out.at[pl.ds(base + i * CHUNK, CHUNK)], osem[i % 2])

        for i in sorted(pending_o):
            pending_o[i].wait()

    return _emb_lookup


_emb_split = _make_emb_lookup(N_TOK // NSPLIT)


def kernel(input_ids, position_ids, word_embeddings, position_embeddings):
    wids = jnp.swapaxes(input_ids, 0, 1).reshape(NSPLIT, N_TOK // NSPLIT)
    pids = jnp.swapaxes(position_ids, 0, 1).reshape(NSPLIT, N_TOK // NSPLIT)
    full = jnp.zeros((S, B, H), jnp.float32)
    for k in range(NSPLIT):
        part = _emb_split(word_embeddings, position_embeddings, wids[k], pids[k])
        part = part.reshape(S // NSPLIT, B, H)
        full = lax.dynamic_update_slice(full, part, (k * (S // NSPLIT), 0, 0))
    return full


# speedup vs baseline: 1.2183x; 1.2183x over previous
import functools

import jax
import jax.numpy as jnp
from jax import lax
from jax.experimental import pallas as pl
from jax.experimental.pallas import tpu as pltpu
from jax.experimental.pallas import tpu_sc as plsc

B = 4
S = 2048
VOCAB = 100000
MAX_POS = 2048
H = 1024

NC = 2
NS = 16
NW = NC * NS
N_TOK = B * S
TOK_PER_W = N_TOK // NW
CHUNK = 16
NCHUNK = TOK_PER_W // CHUNK
LANES = 16
KPER = H // LANES

_mesh = plsc.VectorSubcoreMesh(core_axis_name="c", subcore_axis_name="s")


@functools.partial(
    pl.kernel,
    mesh=_mesh,
    out_type=jax.ShapeDtypeStruct((N_TOK, H), jnp.float32),
    scratch_types=[
        pltpu.VMEM((TOK_PER_W,), jnp.int32),
        pltpu.VMEM((TOK_PER_W,), jnp.int32),
        pltpu.VMEM((4, CHUNK, H), jnp.float32),
        pltpu.VMEM((3, CHUNK, H), jnp.float32),
        pltpu.SemaphoreType.DMA,
        pltpu.SemaphoreType.DMA,
        pltpu.SemaphoreType.DMA,
        pltpu.SemaphoreType.DMA,
        pltpu.SemaphoreType.DMA,
        pltpu.SemaphoreType.DMA,
        pltpu.SemaphoreType.DMA,
        pltpu.SemaphoreType.DMA,
    ],
)
def _emb_lookup(wtab, ptab, wids, pids, out, widx_v, pidx_v, wbuf, pbuf,
                wsem0, wsem1, wsem2, psem0, psem1, psem2, osem0, osem1):
    wsem = (wsem0, wsem1, wsem2)
    psem = (psem0, psem1, psem2)
    osem = (osem0, osem1)
    w = lax.axis_index("s") * NC + lax.axis_index("c")
    base = w * TOK_PER_W

    pltpu.sync_copy(wids.at[pl.ds(base, TOK_PER_W)], widx_v)
    pltpu.sync_copy(pids.at[pl.ds(base, TOK_PER_W)], pidx_v)

    def issue_gather(i):
        wcopy = pltpu.async_copy(
            wtab.at[widx_v.at[pl.ds(i * CHUNK, CHUNK)]], wbuf.at[i % 4],
            wsem[i % 3])
        pcopy = pltpu.async_copy(
            ptab.at[pidx_v.at[pl.ds(i * CHUNK, CHUNK)]], pbuf.at[i % 3],
            psem[i % 3])
        return wcopy, pcopy

    pending_g = {0: issue_gather(0), 1: issue_gather(1)}
    pending_o = {}

    for i in range(NCHUNK):
        ws, ps = i % 4, i % 3
        if i + 2 < NCHUNK:
            if i - 2 in pending_o:
                pending_o.pop(i - 2).wait()
            pending_g[i + 2] = issue_gather(i + 2)
        wc, pc = pending_g.pop(i)
        wc.wait()
        pc.wait()

        def row_body(r, _, _ws=ws, _ps=ps):
            for k in range(KPER):
                x = pbuf[_ps, r, pl.ds(k * LANES, LANES)]
                plsc.addupdate(wbuf.at[_ws, r, pl.ds(k * LANES, LANES)], x)
            return 0

        lax.fori_loop(0, CHUNK, row_body, 0, unroll=False)

        pending_o[i] = pltpu.async_copy(
            wbuf.at[ws], out.at[pl.ds(base + i * CHUNK, CHUNK)], osem[i % 2])

    for i in sorted(pending_o):
        pending_o[i].wait()


def kernel(input_ids, position_ids, word_embeddings, position_embeddings):
    wids = jnp.swapaxes(input_ids, 0, 1).reshape(-1)
    pids = jnp.swapaxes(position_ids, 0, 1).reshape(-1)
    out = _emb_lookup(word_embeddings, position_embeddings, wids, pids)
    return out.reshape(S, B, H)


# speedup vs baseline: 1.2238x; 1.0045x over previous
import functools

import jax
import jax.numpy as jnp
from jax import lax
from jax.experimental import pallas as pl
from jax.experimental.pallas import tpu as pltpu
from jax.experimental.pallas import tpu_sc as plsc

B = 4
S = 2048
VOCAB = 100000
MAX_POS = 2048
H = 1024

NC = 2
NS = 16
NW = NC * NS
N_TOK = B * S
TOK_PER_W = N_TOK // NW
CHUNK = 16
NCHUNK = TOK_PER_W // CHUNK
LANES = 16
KPER = H // LANES

_mesh = plsc.VectorSubcoreMesh(core_axis_name="c", subcore_axis_name="s")


@functools.partial(
    pl.kernel,
    mesh=_mesh,
    out_type=jax.ShapeDtypeStruct((N_TOK, H), jnp.float32),
    scratch_types=[
        pltpu.VMEM((TOK_PER_W,), jnp.int32),
        pltpu.VMEM((TOK_PER_W,), jnp.int32),
        pltpu.VMEM((4, CHUNK, H), jnp.float32),
        pltpu.VMEM((3, CHUNK, H), jnp.float32),
        pltpu.SemaphoreType.DMA,
        pltpu.SemaphoreType.DMA,
        pltpu.SemaphoreType.DMA,
        pltpu.SemaphoreType.DMA,
        pltpu.SemaphoreType.DMA,
        pltpu.SemaphoreType.DMA,
        pltpu.SemaphoreType.DMA,
        pltpu.SemaphoreType.DMA,
    ],
)
def _emb_lookup(wtab, ptab, wids, pids, out, widx_v, pidx_v, wbuf, pbuf,
                wsem0, wsem1, wsem2, psem0, psem1, psem2, osem0, osem1):
    wsem = (wsem0, wsem1, wsem2)
    psem = (psem0, psem1, psem2)
    osem = (osem0, osem1)
    w = lax.axis_index("s") * NC + lax.axis_index("c")
    base = w * TOK_PER_W

    pltpu.sync_copy(wids.at[pl.ds(base, TOK_PER_W)], widx_v)
    pltpu.sync_copy(pids.at[pl.ds(base, TOK_PER_W)], pidx_v)

    def issue_gather(i):
        wcopy = pltpu.async_copy(
            wtab.at[widx_v.at[pl.ds(i * CHUNK, CHUNK)]], wbuf.at[i % 4],
            wsem[i % 3])
        pcopy = pltpu.async_copy(
            ptab.at[pidx_v.at[pl.ds(i * CHUNK, CHUNK)]], pbuf.at[i % 3],
            psem[i % 3])
        return wcopy, pcopy

    pending_g = {0: issue_gather(0), 1: issue_gather(1)}
    pending_o = {}

    for i in range(NCHUNK):
        ws, ps = i % 4, i % 3
        if i + 2 < NCHUNK:
            if i - 2 in pending_o:
                pending_o.pop(i - 2).wait()
            pending_g[i + 2] = issue_gather(i + 2)
        wc, pc = pending_g.pop(i)
        wc.wait()
        pc.wait()

        def row_body(r, _, _ws=ws, _ps=ps):
            for k in range(KPER):
                x = pbuf[_ps, r, pl.ds(k * LANES, LANES)]
                plsc.addupdate(wbuf.at[_ws, r, pl.ds(k * LANES, LANES)], x)
            return 0

        lax.fori_loop(0, CHUNK, row_body, 0, unroll=2)

        pending_o[i] = pltpu.async_copy(
            wbuf.at[ws], out.at[pl.ds(base + i * CHUNK, CHUNK)], osem[i % 2])

    for i in sorted(pending_o):
        pending_o[i].wait()


def kernel(input_ids, position_ids, word_embeddings, position_embeddings):
    wids = jnp.swapaxes(input_ids, 0, 1).reshape(-1)
    pids = jnp.swapaxes(position_ids, 0, 1).reshape(-1)
    out = _emb_lookup(word_embeddings, position_embeddings, wids, pids)
    return out.reshape(S, B, H)
